# baseline (speedup 1.0000x reference)
import functools

import jax
import jax.numpy as jnp
from jax import lax
from jax.experimental import pallas as pl
from jax.experimental.pallas import tpu as pltpu
from jax.experimental.pallas import tpu_sc as plsc

OH = OW = 100
XPAD = 304
NCROP = 128
NW = 32
NPAIR = 16
TMAX = 1600
GSLEN = 288
ACC = OH * XPAD
XROW = 384
KCH = XPAD // 16


def _build_meta(patch_boxes, B, H, W):
    P = patch_boxes.shape[1]
    boxes = patch_boxes.astype(jnp.float32).reshape(NCROP, 4)
    ymin, xmin, ph, pw = boxes[:, 0], boxes[:, 1], boxes[:, 2], boxes[:, 3]
    iy = jnp.arange(OH, dtype=jnp.float32) + 0.5
    ix = jnp.arange(OW, dtype=jnp.float32) + 0.5
    rel_y = iy[None, :] * ph[:, None] / OH - 0.5
    rel_x = ix[None, :] * pw[:, None] / OW - 0.5
    y0f = jnp.floor(rel_y)
    x0f = jnp.floor(rel_x)
    wy = rel_y - y0f
    wx = rel_x - x0f
    y0 = jnp.clip(y0f, 0.0, ph[:, None] - 1.0)
    y1 = jnp.clip(y0f + 1.0, 0.0, ph[:, None] - 1.0)
    x0 = jnp.clip(x0f, 0.0, pw[:, None] - 1.0)
    x1 = jnp.clip(x0f + 1.0, 0.0, pw[:, None] - 1.0)
    ay0 = jnp.clip(ymin[:, None] + y0, 0, H - 1).astype(jnp.int32)
    ay1 = jnp.clip(ymin[:, None] + y1, 0, H - 1).astype(jnp.int32)
    ax0 = jnp.clip(xmin[:, None] + x0, 0, W - 1).astype(jnp.int32)
    ax1 = jnp.clip(xmin[:, None] + x1, 0, W - 1).astype(jnp.int32)
    c3 = jnp.arange(3, dtype=jnp.int32)
    same = ax1 == ax0
    wx_eff = jnp.where(same, 0.0, wx)
    neg = (same & (ax0 > 0)).astype(jnp.int32) << 12
    xip = ((ax0[:, :, None] * 3 + c3) +
           neg[:, :, None]).reshape(NCROP, 3 * OW)
    zpad = jnp.zeros((NCROP, XPAD - 3 * OW), jnp.int32)
    ximeta = jnp.concatenate([xip, zpad], axis=1)
    wxv = jnp.concatenate(
        [jnp.repeat(wx_eff, 3, axis=1), zpad.astype(jnp.float32)], axis=1)
    fuse = (ay1 // 8) == (ay0 // 8)
    ays = jnp.stack([ay0, ay1], axis=-1)
    zf = jnp.zeros_like(wy)
    zi = jnp.zeros_like(ay0)
    ws = jnp.stack([1.0 - wy, wy], axis=-1)
    ws1 = jnp.stack([jnp.where(fuse, wy, 0.0), zf], axis=-1)
    dd = jnp.stack([jnp.where(fuse, ay1 - ay0, 0), zi], axis=-1)
    valid = jnp.stack([fuse | True, ~fuse], axis=-1)
    crop = jnp.arange(NCROP, dtype=jnp.int32)[:, None, None]
    c_local = crop % P
    batch = crop // P
    irow = jnp.arange(OH, dtype=jnp.int32)[None, :, None]
    par = (ays // 8) % 2
    pair = ays // 32
    tile = batch * 2 + par
    slot = ((ays // 16) % 2) * 8 + (ays % 8)
    packed = slot | (dd << 4) | (c_local << 5) | (irow << 8)
    key32 = par * NPAIR + pair
    onehot = ((key32[..., None] == jnp.arange(
        2 * NPAIR, dtype=jnp.int32)) & valid[..., None]).astype(jnp.int8)
    cum = jnp.cumsum(onehot, axis=1, dtype=jnp.int8) - onehot
    rank_i = jnp.sum(
        (cum * onehot).astype(jnp.int32), axis=-1)
    cnt_cs = (cum[:, -1] + onehot[:, -1]).astype(jnp.int32)
    cnt_combo = cnt_cs.reshape(B, P * 2, 2 * NPAIR)
    off_combo = jnp.cumsum(cnt_combo, axis=1) - cnt_combo
    cnt_bucket = cnt_combo.sum(axis=1)
    padded_t = cnt_bucket.reshape(NW, NPAIR)
    pad_off = jnp.cumsum(padded_t, axis=1) - padded_t
    ocs = off_combo.reshape(B, P, 2, 2 * NPAIR).reshape(NCROP, 1, 2,
                                                        2 * NPAIR)
    oh32 = onehot.astype(jnp.int32)
    off1 = jnp.sum(ocs * oh32, axis=-1)
    po_c = jnp.broadcast_to(
        pad_off.reshape(B, 1, 2 * NPAIR),
        (B, P, 2 * NPAIR)).reshape(NCROP, 1, 1, 2 * NPAIR)
    off2 = jnp.sum(po_c * oh32, axis=-1)
    pos = off1 + off2 + rank_i
    tile_f = jnp.where(valid, tile, -1).reshape(-1)
    pos_f = pos.reshape(-1)
    packed_f = jnp.broadcast_to(packed, ays.shape).reshape(-1)
    w0bits_f = jax.lax.bitcast_convert_type(ws, jnp.int32).reshape(-1)
    w1bits_f = jax.lax.bitcast_convert_type(ws1, jnp.int32).reshape(-1)
    taps5 = jnp.stack([tile_f, pos_f, packed_f, w0bits_f, w1bits_f]).reshape(
        5, 64, 400)
    gs = jnp.concatenate(
        [pad_off, pad_off[:, -1:] + padded_t[:, -1:]], axis=1)
    gs_splat = jnp.broadcast_to(gs[:, :, None], (NW, NPAIR + 1, 16))
    gs_splat = jnp.concatenate(
        [gs_splat.reshape(NW, (NPAIR + 1) * 16),
         jnp.zeros((NW, GSLEN - (NPAIR + 1) * 16), jnp.int32)], axis=1)
    return ximeta, wxv, taps5, gs_splat


@functools.cache
def _sc_scatter_fn():
    return pl.kernel(
        _sc_scatter_body,
        out_type=jax.ShapeDtypeStruct((NW, OH, XPAD), jnp.float32),
        mesh=plsc.VectorSubcoreMesh(core_axis_name="c", subcore_axis_name="s"),
        compiler_params=pltpu.CompilerParams(needs_layout_passes=False),
        scratch_types=[
            pltpu.VMEM((8, XPAD), jnp.int32),
            pltpu.VMEM((8, XPAD), jnp.float32),
            pltpu.VMEM((13, 128), jnp.int32),
            pltpu.VMEM((13, 128), jnp.float32),
            pltpu.VMEM((13, 128), jnp.float32),
            pltpu.VMEM((5, 8, 400), jnp.int32),
            pltpu.VMEM((GSLEN,), jnp.int32),
            pltpu.VMEM((32, 1536), jnp.float32),
            pltpu.VMEM((OH, XPAD), jnp.float32),
            pltpu.SemaphoreType.DMA,
        ],
    )


def _sc_scatter_body(g2, ximeta, wxv, taps5, gs, out, xmi_v, wx_v,
                     tapi_v, tapw_v, tapw1_v, stage_v, gs_v, blk_v, acc_v,
                     sem):
    wid = lax.axis_index("s") * 2 + lax.axis_index("c")
    batch = wid // 2
    par = wid % 2
    pltpu.sync_copy(gs.at[wid], gs_v)
    for c in range(8):
        pltpu.sync_copy(ximeta.at[batch * 8 + c], xmi_v.at[c])
        pltpu.sync_copy(wxv.at[batch * 8 + c], wx_v.at[c])

    zeros16 = jnp.zeros((16,), jnp.float32)
    izeros16 = jnp.zeros((16,), jnp.int32)

    def zbody(i, carry):
        for k in range(KCH):
            acc_v[i, pl.ds(k * 16, 16)] = zeros16
        return carry

    lax.fori_loop(0, OH, zbody, 0)

    def ztap(i, carry):
        for k in range(8):
            tapi_v[i, pl.ds(k * 16, 16)] = izeros16
            tapw_v[i, pl.ds(k * 16, 16)] = zeros16
            tapw1_v[i, pl.ds(k * 16, 16)] = zeros16
        return carry

    lax.fori_loop(0, 13, ztap, 0)

    widv = jnp.full((16,), wid, jnp.int32)
    for mega in range(8):
        for f in range(5):
            pltpu.sync_copy(
                taps5.at[f, pl.ds(mega * 8, 8)], stage_v.at[f])

        def srow(r, carry):
            def scol(j, c2):
                tl = stage_v[0, r, pl.ds(j * 16, 16)]
                ps = stage_v[1, r, pl.ds(j * 16, 16)]
                pk = stage_v[2, r, pl.ds(j * 16, 16)]
                w0b = stage_v[3, r, pl.ds(j * 16, 16)]
                w1b = stage_v[4, r, pl.ds(j * 16, 16)]
                m = tl == widv
                plsc.store_scatter(tapi_v, [ps >> 7, ps & 127], pk, mask=m)
                plsc.store_scatter(
                    tapw_v, [ps >> 7, ps & 127],
                    plsc.bitcast(w0b, jnp.float32), mask=m)
                plsc.store_scatter(
                    tapw1_v, [ps >> 7, ps & 127],
                    plsc.bitcast(w1b, jnp.float32), mask=m)
                return c2

            lax.fori_loop(0, 25, scol, 0)
            return carry

        lax.fori_loop(0, 8, srow, 0)

    iota = lax.iota(jnp.int32, 16)

    def rowbase(pp, half):
        return batch * 512 + 8 * (4 * pp + 2 * half + par)

    def issue(pp, boff):
        pltpu.async_copy(
            g2.at[pl.ds(rowbase(pp, 0), 8)], blk_v.at[pl.ds(boff, 8)], sem)
        pltpu.async_copy(
            g2.at[pl.ds(rowbase(pp, 1), 8)], blk_v.at[pl.ds(boff + 8, 8)],
            sem)

    def drain(pp, boff):
        pltpu.make_async_copy(
            g2.at[pl.ds(rowbase(pp, 0), 8)], blk_v.at[pl.ds(boff, 8)],
            sem).wait()
        pltpu.make_async_copy(
            g2.at[pl.ds(rowbase(pp, 1), 8)], blk_v.at[pl.ds(boff + 8, 8)],
            sem).wait()

    issue(0, 0)

    def pair_body(pp, carry):
        boff = (pp % 2) * 16
        drain(pp, boff)

        @pl.when(pp < NPAIR - 1)
        def _prefetch():
            issue(pp + 1, 16 - boff)

        t_lo = gs_v[pl.ds(pp * 16, 16)][0]
        t_hi = gs_v[pl.ds((pp + 1) * 16, 16)][0]

        def tap_body(t, c2):
            td = jnp.full((16,), t >> 7, jnp.int32)
            tm = jnp.full((16,), t & 127, jnp.int32)
            tw_s = plsc.load_gather(tapi_v, [td, tm])
            w0_s = plsc.load_gather(tapw_v, [td, tm])
            w1_s = plsc.load_gather(tapw1_v, [td, tm])
            slot0 = (tw_s & 15) + boff
            slot1 = slot0 + ((tw_s >> 4) & 1)
            cv = (tw_s >> 5) & 7
            irv = tw_s >> 8
            for k in range(KCH):
                kv = iota + (k * 16)
                xip = plsc.load_gather(xmi_v, [cv, kv])
                xi0 = xip & 4095
                xi1 = xi0 + 3 - ((xip >> 11) & 2) * 3
                wx = plsc.load_gather(wx_v, [cv, kv])
                v00 = plsc.load_gather(blk_v, [slot0, xi0])
                v01 = plsc.load_gather(blk_v, [slot0, xi1])
                v10 = plsc.load_gather(blk_v, [slot1, xi0])
                v11 = plsc.load_gather(blk_v, [slot1, xi1])
                h0 = v00 + wx * (v01 - v00)
                h1 = v10 + wx * (v11 - v10)
                plsc.addupdate_scatter(
                    acc_v, [irv, kv], w0_s * h0 + w1_s * h1)
            return c2

        lax.fori_loop(t_lo, t_hi, tap_body, 0)
        return carry

    lax.fori_loop(0, NPAIR, pair_body, 0)
    pltpu.sync_copy(acc_v, out.at[wid])


def _tc_reduce(parts):
    def body(x_ref, o_ref):
        o_ref[...] = jnp.sum(x_ref[...], axis=0)

    return pl.pallas_call(
        body,
        out_shape=jax.ShapeDtypeStruct((OH, XPAD), jnp.float32),
    )(parts)


def kernel(gradients, patch_boxes, transform_decisions):
    B, H, W, C = gradients.shape
    ximeta, wxv, taps5, gs = _build_meta(patch_boxes, B, H, W)
    gview = gradients.reshape(B * H, W * C)
    parts = _sc_scatter_fn()(gview, ximeta, wxv, taps5, gs)
    total = _tc_reduce(parts)
    return total[:OH, :3 * OW].reshape(OH, OW, 3)


# speedup vs baseline: 1.1626x; 1.1626x over previous
import functools

import jax
import jax.numpy as jnp
from jax import lax
from jax.experimental import pallas as pl
from jax.experimental.pallas import tpu as pltpu
from jax.experimental.pallas import tpu_sc as plsc

OH = OW = 100
XPAD = 304
NCROP = 128
NW = 32
NPAIR = 16
TMAX = 1600
GSLEN = 288
ACC = OH * XPAD
XROW = 384
KCH = XPAD // 16


def _build_meta(patch_boxes, B, H, W):
    P = patch_boxes.shape[1]
    boxes = patch_boxes.astype(jnp.float32).reshape(NCROP, 4)
    ymin, xmin, ph, pw = boxes[:, 0], boxes[:, 1], boxes[:, 2], boxes[:, 3]
    iy = jnp.arange(OH, dtype=jnp.float32) + 0.5
    ix = jnp.arange(OW, dtype=jnp.float32) + 0.5
    rel_y = iy[None, :] * ph[:, None] / OH - 0.5
    rel_x = ix[None, :] * pw[:, None] / OW - 0.5
    y0f = jnp.floor(rel_y)
    x0f = jnp.floor(rel_x)
    wy = rel_y - y0f
    wx = rel_x - x0f
    y0 = jnp.clip(y0f, 0.0, ph[:, None] - 1.0)
    y1 = jnp.clip(y0f + 1.0, 0.0, ph[:, None] - 1.0)
    x0 = jnp.clip(x0f, 0.0, pw[:, None] - 1.0)
    x1 = jnp.clip(x0f + 1.0, 0.0, pw[:, None] - 1.0)
    ay0 = jnp.clip(ymin[:, None] + y0, 0, H - 1).astype(jnp.int32)
    ay1 = jnp.clip(ymin[:, None] + y1, 0, H - 1).astype(jnp.int32)
    ax0 = jnp.clip(xmin[:, None] + x0, 0, W - 1).astype(jnp.int32)
    ax1 = jnp.clip(xmin[:, None] + x1, 0, W - 1).astype(jnp.int32)
    c3 = jnp.arange(3, dtype=jnp.int32)
    xi0 = (ax0[:, :, None] * 3 + c3).reshape(NCROP, 3 * OW)
    xi1 = (ax1[:, :, None] * 3 + c3).reshape(NCROP, 3 * OW)
    zpad = jnp.zeros((NCROP, XPAD - 3 * OW), jnp.int32)
    ximeta = jnp.concatenate(
        [xi0, zpad, xi1, zpad], axis=1)
    wxv = jnp.concatenate(
        [jnp.repeat(wx, 3, axis=1), zpad.astype(jnp.float32)], axis=1)
    fuse = (ay1 // 8) == (ay0 // 8)
    ays = jnp.stack([ay0, ay1], axis=-1)
    zf = jnp.zeros_like(wy)
    zi = jnp.zeros_like(ay0)
    ws = jnp.stack([1.0 - wy, wy], axis=-1)
    ws1 = jnp.stack([jnp.where(fuse, wy, 0.0), zf], axis=-1)
    dd = jnp.stack([jnp.where(fuse, ay1 - ay0, 0), zi], axis=-1)
    valid = jnp.stack([fuse | True, ~fuse], axis=-1)
    crop = jnp.arange(NCROP, dtype=jnp.int32)[:, None, None]
    c_local = crop % P
    batch = crop // P
    irow = jnp.arange(OH, dtype=jnp.int32)[None, :, None]
    par = (ays // 8) % 2
    pair = ays // 32
    tile = batch * 2 + par
    slot = ((ays // 16) % 2) * 8 + (ays % 8)
    packed = slot | (dd << 4) | (c_local << 5) | (irow << 8)
    key32 = par * NPAIR + pair
    onehot = ((key32[..., None] == jnp.arange(
        2 * NPAIR, dtype=jnp.int32)) & valid[..., None]).astype(jnp.int8)
    cum = jnp.cumsum(onehot, axis=1, dtype=jnp.int8) - onehot
    rank_i = jnp.sum(
        (cum * onehot).astype(jnp.int32), axis=-1)
    cnt_cs = (cum[:, -1] + onehot[:, -1]).astype(jnp.int32)
    cnt_combo = cnt_cs.reshape(B, P * 2, 2 * NPAIR)
    off_combo = jnp.cumsum(cnt_combo, axis=1) - cnt_combo
    cnt_bucket = cnt_combo.sum(axis=1)
    padded_t = cnt_bucket.reshape(NW, NPAIR)
    pad_off = jnp.cumsum(padded_t, axis=1) - padded_t
    ocs = off_combo.reshape(B, P, 2, 2 * NPAIR).reshape(NCROP, 1, 2,
                                                        2 * NPAIR)
    oh32 = onehot.astype(jnp.int32)
    off1 = jnp.sum(ocs * oh32, axis=-1)
    po_c = jnp.broadcast_to(
        pad_off.reshape(B, 1, 2 * NPAIR),
        (B, P, 2 * NPAIR)).reshape(NCROP, 1, 1, 2 * NPAIR)
    off2 = jnp.sum(po_c * oh32, axis=-1)
    pos = off1 + off2 + rank_i
    tile_f = jnp.where(valid, tile, -1).reshape(-1)
    pos_f = pos.reshape(-1)
    packed_f = jnp.broadcast_to(packed, ays.shape).reshape(-1)
    w0bits_f = jax.lax.bitcast_convert_type(ws, jnp.int32).reshape(-1)
    w1bits_f = jax.lax.bitcast_convert_type(ws1, jnp.int32).reshape(-1)
    taps5 = jnp.stack([tile_f, pos_f, packed_f, w0bits_f, w1bits_f]).reshape(
        5, 64, 400)
    gs = jnp.concatenate(
        [pad_off, pad_off[:, -1:] + padded_t[:, -1:]], axis=1)
    gs_splat = jnp.broadcast_to(gs[:, :, None], (NW, NPAIR + 1, 16))
    gs_splat = jnp.concatenate(
        [gs_splat.reshape(NW, (NPAIR + 1) * 16),
         jnp.zeros((NW, GSLEN - (NPAIR + 1) * 16), jnp.int32)], axis=1)
    return ximeta, wxv, taps5, gs_splat


@functools.cache
def _sc_scatter_fn():
    return pl.kernel(
        _sc_scatter_body,
        out_type=jax.ShapeDtypeStruct((NW, OH, XPAD), jnp.float32),
        mesh=plsc.VectorSubcoreMesh(core_axis_name="c", subcore_axis_name="s"),
        compiler_params=pltpu.CompilerParams(needs_layout_passes=False),
        scratch_types=[
            pltpu.VMEM((8, 608), jnp.int32),
            pltpu.VMEM((8, XPAD), jnp.float32),
            pltpu.VMEM((13, 128), jnp.int32),
            pltpu.VMEM((13, 128), jnp.float32),
            pltpu.VMEM((13, 128), jnp.float32),
            pltpu.VMEM((5, 8, 400), jnp.int32),
            pltpu.VMEM((GSLEN,), jnp.int32),
            pltpu.VMEM((32, 1536), jnp.float32),
            pltpu.VMEM((OH, XPAD), jnp.float32),
            pltpu.SemaphoreType.DMA,
        ],
    )


def _sc_scatter_body(g2, ximeta, wxv, taps5, gs, out, xmi_v, wx_v,
                     tapi_v, tapw_v, tapw1_v, stage_v, gs_v, blk_v, acc_v,
                     sem):
    wid = lax.axis_index("s") * 2 + lax.axis_index("c")
    batch = wid // 2
    par = wid % 2
    pltpu.sync_copy(gs.at[wid], gs_v)
    for c in range(8):
        pltpu.sync_copy(ximeta.at[batch * 8 + c], xmi_v.at[c])
        pltpu.sync_copy(wxv.at[batch * 8 + c], wx_v.at[c])

    zeros16 = jnp.zeros((16,), jnp.float32)
    izeros16 = jnp.zeros((16,), jnp.int32)

    def zbody(i, carry):
        for k in range(KCH):
            acc_v[i, pl.ds(k * 16, 16)] = zeros16
        return carry

    lax.fori_loop(0, OH, zbody, 0)

    def ztap(i, carry):
        for k in range(8):
            tapi_v[i, pl.ds(k * 16, 16)] = izeros16
            tapw_v[i, pl.ds(k * 16, 16)] = zeros16
            tapw1_v[i, pl.ds(k * 16, 16)] = zeros16
        return carry

    lax.fori_loop(0, 13, ztap, 0)

    widv = jnp.full((16,), wid, jnp.int32)
    for f in range(5):
        pltpu.sync_copy(
            taps5.at[f, pl.ds(4 * batch, 4)], stage_v.at[f, pl.ds(0, 4)])

    def srow(r, carry):
        def scol(j, c2):
            tl = stage_v[0, r, pl.ds(j * 16, 16)]
            ps = stage_v[1, r, pl.ds(j * 16, 16)]
            pk = stage_v[2, r, pl.ds(j * 16, 16)]
            w0b = stage_v[3, r, pl.ds(j * 16, 16)]
            w1b = stage_v[4, r, pl.ds(j * 16, 16)]
            m = tl == widv
            plsc.store_scatter(tapi_v, [ps >> 7, ps & 127], pk, mask=m)
            plsc.store_scatter(
                tapw_v, [ps >> 7, ps & 127],
                plsc.bitcast(w0b, jnp.float32), mask=m)
            plsc.store_scatter(
                tapw1_v, [ps >> 7, ps & 127],
                plsc.bitcast(w1b, jnp.float32), mask=m)
            return c2

        lax.fori_loop(0, 25, scol, 0)
        return carry

    lax.fori_loop(0, 4, srow, 0)

    iota = lax.iota(jnp.int32, 16)

    def rowbase(pp, half):
        return batch * 512 + 8 * (4 * pp + 2 * half + par)

    def issue(pp, boff):
        pltpu.async_copy(
            g2.at[pl.ds(rowbase(pp, 0), 8)], blk_v.at[pl.ds(boff, 8)], sem)
        pltpu.async_copy(
            g2.at[pl.ds(rowbase(pp, 1), 8)], blk_v.at[pl.ds(boff + 8, 8)],
            sem)

    def drain(pp, boff):
        pltpu.make_async_copy(
            g2.at[pl.ds(rowbase(pp, 0), 8)], blk_v.at[pl.ds(boff, 8)],
            sem).wait()
        pltpu.make_async_copy(
            g2.at[pl.ds(rowbase(pp, 1), 8)], blk_v.at[pl.ds(boff + 8, 8)],
            sem).wait()

    issue(0, 0)

    def pair_body(pp, carry):
        boff = (pp % 2) * 16
        drain(pp, boff)

        @pl.when(pp < NPAIR - 1)
        def _prefetch():
            issue(pp + 1, 16 - boff)

        t_lo = gs_v[pl.ds(pp * 16, 16)][0]
        t_hi = gs_v[pl.ds((pp + 1) * 16, 16)][0]

        def tap_body(t, c2):
            td = jnp.full((16,), t >> 7, jnp.int32)
            tm = jnp.full((16,), t & 127, jnp.int32)
            tw_s = plsc.load_gather(tapi_v, [td, tm])
            w0_s = plsc.load_gather(tapw_v, [td, tm])
            w1_s = plsc.load_gather(tapw1_v, [td, tm])
            slot0 = (tw_s & 15) + boff
            slot1 = slot0 + ((tw_s >> 4) & 1)
            cv = (tw_s >> 5) & 7
            irv = tw_s >> 8
            for k in range(KCH):
                kv = iota + (k * 16)
                xi0 = plsc.load_gather(xmi_v, [cv, kv])
                xi1 = plsc.load_gather(xmi_v, [cv, kv + XPAD])
                wx = plsc.load_gather(wx_v, [cv, kv])
                v00 = plsc.load_gather(blk_v, [slot0, xi0])
                v01 = plsc.load_gather(blk_v, [slot0, xi1])
                v10 = plsc.load_gather(blk_v, [slot1, xi0])
                v11 = plsc.load_gather(blk_v, [slot1, xi1])
                h0 = v00 + wx * (v01 - v00)
                h1 = v10 + wx * (v11 - v10)
                plsc.addupdate_scatter(
                    acc_v, [irv, kv], w0_s * h0 + w1_s * h1)
            return c2

        lax.fori_loop(t_lo, t_hi, tap_body, 0)
        return carry

    lax.fori_loop(0, NPAIR, pair_body, 0)
    pltpu.sync_copy(acc_v, out.at[wid])


def _tc_reduce(parts):
    def body(x_ref, o_ref):
        o_ref[...] = jnp.sum(x_ref[...], axis=0)

    return pl.pallas_call(
        body,
        out_shape=jax.ShapeDtypeStruct((OH, XPAD), jnp.float32),
    )(parts)


def kernel(gradients, patch_boxes, transform_decisions):
    B, H, W, C = gradients.shape
    ximeta, wxv, taps5, gs = _build_meta(patch_boxes, B, H, W)
    gview = gradients.reshape(B * H, W * C)
    parts = _sc_scatter_fn()(gview, ximeta, wxv, taps5, gs)
    total = _tc_reduce(parts)
    return total[:OH, :3 * OW].reshape(OH, OW, 3)
